# baseline (speedup 1.0000x reference)
import functools

import jax
import jax.numpy as jnp
from jax import lax
from jax.experimental import pallas as pl
from jax.experimental.pallas import tpu as pltpu
from jax.experimental.pallas import tpu_sc as plsc

_D = 64
_DP = 128
_NE = 1024
_NB = 16
_NBH = 8
_BM = 1024
_ROWS = _NB * _BM
_HROWS = _NBH * _BM

_NC, _NS = 2, 16
_NW = _NC * _NS
_PW = _HROWS // _NW
_CH = _PW // 128

_MM_PRECISION = lax.Precision.DEFAULT


def _dist_argmin_body(xt_ref, e_ref, ind_ref, isc_ref, dsum_ref, etp_ref):
    i = pl.program_id(0)
    xt = xt_ref[0]
    e = e_ref[...]
    a = jnp.sum(xt * xt, axis=0, keepdims=True)
    bt = jnp.sum(e * e, axis=0).reshape(_NE, 1)
    em2 = -2.0 * e
    m2 = lax.dot_general(em2, xt, (((0,), (0,)), ((), ())),
                         preferred_element_type=jnp.float32,
                         precision=_MM_PRECISION)
    dist = (a + m2) + bt
    kbits = lax.bitcast_convert_type(dist, jnp.int32)
    abits = lax.bitcast_convert_type(a, jnp.int32)
    row = lax.broadcasted_iota(jnp.int32, dist.shape, 0)
    key = (kbits - abits) * 1024 + row
    kmin = jnp.min(key, axis=0)
    ind = jnp.bitwise_and(kmin, 1023)
    ind_ref[0, 0, :] = ind
    ind8 = ind.reshape(8, 128)
    for q in range(4):
        isc_ref[0, 8 * q:8 * q + 2, :] = ind8[2 * q:2 * q + 2]
    dmin = lax.bitcast_convert_type(
        jnp.right_shift(kmin, 10) + abits[0], jnp.float32)

    @pl.when(i == 0)
    def _():
        dsum_ref[0, 0] = 0.0
        etp_ref[...] = jnp.pad(e.T, ((0, 0), (0, _DP - _D)))

    dsum_ref[0, 0] += jnp.sum(dmin)


def _tc_half(xth, embed):
    return pl.pallas_call(
        _dist_argmin_body,
        grid=(_NBH,),
        in_specs=[
            pl.BlockSpec((1, _D, _BM), lambda i: (i, 0, 0)),
            pl.BlockSpec((_D, _NE), lambda i: (0, 0)),
        ],
        out_specs=[
            pl.BlockSpec((1, 1, _BM), lambda i: (i, 0, 0)),
            pl.BlockSpec((1, 32, 128), lambda i: (i, 0, 0)),
            pl.BlockSpec(memory_space=pltpu.SMEM),
            pl.BlockSpec((_NE, _DP), lambda i: (0, 0)),
        ],
        out_shape=[
            jax.ShapeDtypeStruct((_NBH, 1, _BM), jnp.int32),
            jax.ShapeDtypeStruct((_NBH, 32, 128), jnp.int32),
            jax.ShapeDtypeStruct((1, 1), jnp.float32),
            jax.ShapeDtypeStruct((_NE, _DP), jnp.float32),
        ],
        compiler_params=pltpu.CompilerParams(
            dimension_semantics=("arbitrary",),
        ),
    )(xth, embed)


@functools.cache
def _make_sc_gather():
    @functools.partial(
        pl.kernel,
        out_type=jax.ShapeDtypeStruct((_HROWS, _DP), jnp.float32),
        mesh=plsc.VectorSubcoreMesh(core_axis_name="c", subcore_axis_name="s",
                                    num_cores=_NC, num_subcores=_NS),
        scratch_types=[
            pltpu.VMEM((_CH, 128), jnp.int32),
            pltpu.VMEM((_PW, _DP), jnp.float32),
            [pltpu.SemaphoreType.DMA] * _CH,
            [pltpu.SemaphoreType.DMA] * _CH,
        ],
    )
    def _sc_gather(etp_hbm, idx_hbm, out_hbm, idx_v, rows_v, gsems, wsems):
        wid = lax.axis_index("s") * _NC + lax.axis_index("c")
        b = wid // 4
        q = wid % 4
        pltpu.sync_copy(idx_hbm.at[b, pl.ds(q * 8, _CH)], idx_v)
        gathers = [
            pltpu.async_copy(etp_hbm.at[idx_v.at[c]],
                             rows_v.at[pl.ds(c * 128, 128)], gsems[c])
            for c in range(_CH)
        ]
        writes = []
        for c in range(_CH):
            gathers[c].wait()
            writes.append(pltpu.async_copy(
                rows_v.at[pl.ds(c * 128, 128)],
                out_hbm.at[pl.ds(b * _BM + q * _PW + c * 128, 128)],
                wsems[c]))
        for w in writes:
            w.wait()

    return _sc_gather


def kernel(input, embed):
    xt = input.transpose(0, 2, 1)
    ind_a, isc_a, dsum_a, etp = _tc_half(xt[:_NBH], embed)
    ind_b, isc_b, dsum_b, _ = _tc_half(xt[_NBH:], embed)
    gather = _make_sc_gather()
    q_a = gather(etp, isc_a)
    q_b = gather(etp, isc_b)
    q2 = jnp.concatenate([q_a, q_b], axis=0)
    quantize = q2[:, :_D].reshape(_NB, _BM, _D)
    diff = (dsum_a[0, 0] + dsum_b[0, 0]) / jnp.float32(_ROWS * _D)
    embed_ind = jnp.concatenate([ind_a, ind_b], axis=0).reshape(
        input.shape[:-1])
    return quantize, diff, embed_ind


# speedup vs baseline: 1.3010x; 1.3010x over previous
import functools

import jax
import jax.numpy as jnp
from jax import lax
from jax.experimental import pallas as pl
from jax.experimental.pallas import tpu as pltpu
from jax.experimental.pallas import tpu_sc as plsc

_D = 64
_DP = 128
_NE = 1024
_NB = 16
_BM = 1024
_ROWS = _NB * _BM

_NC, _NS = 2, 16
_NW = _NC * _NS
_PW = _ROWS // _NW
_CH = 4

_MM_PRECISION = lax.Precision.DEFAULT


def _dist_argmin_body(xt_ref, e_ref, ind_ref, isc_ref, dsum_ref, etp_ref):
    i = pl.program_id(0)
    xt = xt_ref[0]
    e = e_ref[...]
    a = jnp.sum(xt * xt, axis=0, keepdims=True)
    bt = jnp.sum(e * e, axis=0).reshape(_NE, 1)
    em2 = -2.0 * e
    m2 = lax.dot_general(em2, xt, (((0,), (0,)), ((), ())),
                         preferred_element_type=jnp.float32,
                         precision=_MM_PRECISION)
    dist = (a + m2) + bt
    kbits = lax.bitcast_convert_type(dist, jnp.int32)
    abits = lax.bitcast_convert_type(a, jnp.int32)
    row = lax.broadcasted_iota(jnp.int32, dist.shape, 0)
    key = (kbits - abits) * 1024 + row
    kmin = jnp.min(key, axis=0)
    ind = jnp.bitwise_and(kmin, 1023)
    ind_ref[0, 0, :] = ind
    ind8 = ind.reshape(8, 128)
    isc_ref[0, 0:4, :] = ind8[0:4]
    isc_ref[0, 8:12, :] = ind8[4:8]
    dmin = lax.bitcast_convert_type(
        jnp.right_shift(kmin, 10) + abits[0], jnp.float32)

    @pl.when(i == 0)
    def _():
        dsum_ref[0, 0] = 0.0
        etp_ref[...] = jnp.pad(e.T, ((0, 0), (0, _DP - _D)))

    dsum_ref[0, 0] += jnp.sum(dmin)


@functools.cache
def _make_sc_gather():
    @functools.partial(
        pl.kernel,
        out_type=jax.ShapeDtypeStruct((_ROWS, _DP), jnp.float32),
        mesh=plsc.VectorSubcoreMesh(core_axis_name="c", subcore_axis_name="s",
                                    num_cores=_NC, num_subcores=_NS),
        scratch_types=[
            pltpu.VMEM((_CH, 128), jnp.int32),
            pltpu.VMEM((_PW, _DP), jnp.float32),
            [pltpu.SemaphoreType.DMA] * _CH,
            [pltpu.SemaphoreType.DMA] * _CH,
        ],
    )
    def _sc_gather(etp_hbm, idx_hbm, out_hbm, idx_v, rows_v, gsems, wsems):
        wid = lax.axis_index("s") * _NC + lax.axis_index("c")
        b = wid // 2
        h = wid % 2
        pltpu.sync_copy(idx_hbm.at[b, pl.ds(h * 8, _CH)], idx_v)
        gathers = [
            pltpu.async_copy(etp_hbm.at[idx_v.at[c]],
                             rows_v.at[pl.ds(c * 128, 128)], gsems[c])
            for c in range(_CH)
        ]
        writes = []
        for c in range(_CH):
            gathers[c].wait()
            writes.append(pltpu.async_copy(
                rows_v.at[pl.ds(c * 128, 128)],
                out_hbm.at[pl.ds(wid * _PW + c * 128, 128)], wsems[c]))
        for w in writes:
            w.wait()

    return _sc_gather


def kernel(input, embed):
    xt = input.transpose(0, 2, 1)
    ind3, isc, dsum, etp = pl.pallas_call(
        _dist_argmin_body,
        grid=(_NB,),
        in_specs=[
            pl.BlockSpec((1, _D, _BM), lambda i: (i, 0, 0)),
            pl.BlockSpec((_D, _NE), lambda i: (0, 0)),
        ],
        out_specs=[
            pl.BlockSpec((1, 1, _BM), lambda i: (i, 0, 0)),
            pl.BlockSpec((1, 16, 128), lambda i: (i, 0, 0)),
            pl.BlockSpec(memory_space=pltpu.SMEM),
            pl.BlockSpec((_NE, _DP), lambda i: (0, 0)),
        ],
        out_shape=[
            jax.ShapeDtypeStruct((_NB, 1, _BM), jnp.int32),
            jax.ShapeDtypeStruct((_NB, 16, 128), jnp.int32),
            jax.ShapeDtypeStruct((1, 1), jnp.float32),
            jax.ShapeDtypeStruct((_NE, _DP), jnp.float32),
        ],
        compiler_params=pltpu.CompilerParams(
            dimension_semantics=("arbitrary",),
        ),
    )(xt, embed)
    q2 = _make_sc_gather()(etp, isc)
    quantize = q2[:, :_D].reshape(_NB, _BM, _D)
    diff = dsum[0, 0] / jnp.float32(_ROWS * _D)
    embed_ind = ind3.reshape(input.shape[:-1])
    return quantize, diff, embed_ind
